# Initial kernel scaffold; baseline (speedup 1.0000x reference)
import jax
import jax.numpy as jnp
from jax.experimental import pallas as pl


def kernel(atom_idx, charge_idx, degree_idx, edge_index, batch, sub_fp, fingerprint, emb_atom, emb_charge, emb_degree, proj_W1, proj_b1, proj_W2, proj_b2, fp_W, fp_b, cat_W0, cat_as0, cat_ad0, cat_b0, cat_p0, cat_pb0, cat_W1, cat_as1, cat_ad1, cat_b1, cat_p1, cat_pb1, cat_W2, cat_as2, cat_ad2, cat_b2, cat_p2, cat_pb2, fpc_W0, fpc_as0, fpc_ad0, fpc_b0, fpc_p0, fpc_pb0, fpc_W1, fpc_as1, fpc_ad1, fpc_b1, fpc_p1, fpc_pb1, fpc_W2, fpc_as2, fpc_ad2, fpc_b2, fpc_p2, fpc_pb2, lin1_W, lin1_b, lin2_W, lin2_b):
    raise NotImplementedError("write your pallas kernel here")


# speedup vs baseline: 1.2591x; 1.2591x over previous
import functools
import jax
import jax.numpy as jnp
from jax.experimental import pallas as pl
from jax.experimental.pallas import tpu as pltpu

NEG = -1e9
B_GR = 64


def _lrelu(x):
    return jnp.where(x >= 0, x, 0.2 * x)


def _mm_body(x_ref, w_ref, b_ref, o_ref, *, act):
    acc = jnp.dot(x_ref[...], w_ref[...], preferred_element_type=jnp.float32)
    acc = acc + b_ref[...]
    if act == 'relu':
        acc = jnp.maximum(acc, 0.0)
    elif act == 'sigmoid':
        acc = jax.nn.sigmoid(acc)
    o_ref[...] = acc


def mm(x, w, b=None, act=None, bm=512):
    M, K = x.shape
    N = w.shape[1]
    if b is None:
        b = jnp.zeros((N,), jnp.float32)
    Mp = ((M + bm - 1) // bm) * bm
    if Mp != M:
        x = jnp.pad(x, ((0, Mp - M), (0, 0)))
    out = pl.pallas_call(
        functools.partial(_mm_body, act=act),
        grid=(Mp // bm,),
        in_specs=[
            pl.BlockSpec((bm, K), lambda i: (i, 0)),
            pl.BlockSpec((K, N), lambda i: (0, 0)),
            pl.BlockSpec((1, N), lambda i: (0, 0)),
        ],
        out_specs=pl.BlockSpec((bm, N), lambda i: (i, 0)),
        out_shape=jax.ShapeDtypeStruct((Mp, N), jnp.float32),
    )(x, w, b.reshape(1, N))
    return out[:M] if Mp != M else out


def _emb_body(ai_ref, ci_ref, di_ref, tab_ref, w1_ref, b1_ref, w2_ref, b2_ref,
              o_ref):
    bm = ai_ref.shape[1]
    ai = ai_ref[0, 0]
    ci = ci_ref[0, 0]
    di = di_ref[0, 0]
    col = jax.lax.broadcasted_iota(jnp.int32, (bm, 384), 1)
    oh = ((col == ai[:, None]) | (col == ci[:, None] + 128)
          | (col == di[:, None] + 256)).astype(jnp.float32)
    embs = jnp.dot(oh, tab_ref[...], preferred_element_type=jnp.float32)
    h = jnp.maximum(jnp.dot(embs, w1_ref[...],
                            preferred_element_type=jnp.float32) + b1_ref[...], 0.0)
    o_ref[...] = jnp.dot(h, w2_ref[...],
                         preferred_element_type=jnp.float32) + b2_ref[...]


def embed_proj(atom_idx, charge_idx, degree_idx, emb_atom, emb_charge,
               emb_degree, W1, b1, W2, b2, bm=512):
    N = atom_idx.shape[0]
    Np = ((N + bm - 1) // bm) * bm
    ai = jnp.pad(atom_idx.astype(jnp.int32), (0, Np - N)).reshape(Np // bm, 1, bm)
    ci = jnp.pad(charge_idx.astype(jnp.int32), (0, Np - N)).reshape(Np // bm, 1, bm)
    di = jnp.pad(degree_idx.astype(jnp.int32), (0, Np - N)).reshape(Np // bm, 1, bm)
    tab = jnp.zeros((384, 384), jnp.float32)
    tab = tab.at[0:100, 0:128].set(emb_atom)
    tab = tab.at[128:144, 128:256].set(emb_charge)
    tab = tab.at[256:264, 256:384].set(emb_degree)
    D = W2.shape[1]
    out = pl.pallas_call(
        _emb_body,
        grid=(Np // bm,),
        in_specs=[
            pl.BlockSpec((1, 1, bm), lambda i: (i, 0, 0)),
            pl.BlockSpec((1, 1, bm), lambda i: (i, 0, 0)),
            pl.BlockSpec((1, 1, bm), lambda i: (i, 0, 0)),
            pl.BlockSpec((384, 384), lambda i: (0, 0)),
            pl.BlockSpec((384, 128), lambda i: (0, 0)),
            pl.BlockSpec((1, 128), lambda i: (0, 0)),
            pl.BlockSpec((128, D), lambda i: (0, 0)),
            pl.BlockSpec((1, D), lambda i: (0, 0)),
        ],
        out_specs=pl.BlockSpec((bm, D), lambda i: (i, 0)),
        out_shape=jax.ShapeDtypeStruct((Np, D), jnp.float32),
    )(ai, ci, di, tab, W1, b1.reshape(1, -1), W2, b2.reshape(1, -1))
    return out[:N]


def gat_full(x, src, dst, emask, W, a_s, a_d, b):
    n = x.shape[0]
    h = mm(x, W)
    aw = jnp.stack([a_s, a_d], axis=1)
    s = h @ aw
    s_src = s[:, 0]
    s_dst = s[:, 1]
    c = _lrelu(jnp.max(s_src) + jnp.max(s_dst))
    alpha = _lrelu(s_src[src] + s_dst[dst])
    e = jnp.exp(alpha - c) * emask
    denom = jax.ops.segment_sum(e, dst, num_segments=n,
                                indices_are_sorted=True) + 1e-16
    out = jax.ops.segment_sum(h[src] * e[:, None], dst, num_segments=n,
                              indices_are_sorted=True)
    return jnp.maximum(out / denom[:, None] + b, 0.0)


def pool_full(x, alive, src, dst, emask, pw, pb, k):
    fitness = jax.nn.sigmoid(x @ pw + pb)
    fb = jax.lax.bitcast_convert_type(fitness, jnp.int32)
    key = jnp.where(alive, fb, jnp.int32(-1))
    lo = jnp.int32(-2)
    hi = jnp.int32(0x40000000)

    def body(i, lohi):
        lo, hi = lohi
        mid = lo + (hi - lo) // 2
        cnt = jnp.sum((key > mid).astype(jnp.int32))
        return jnp.where(cnt >= k, mid + 1, lo), jnp.where(cnt >= k, hi, mid)

    lo, hi = jax.lax.fori_loop(0, 32, body, (lo, hi))
    T = lo
    strict = key > T
    c_strict = jnp.sum(strict.astype(jnp.int32))
    tie = key == T
    rank = jnp.cumsum(tie.astype(jnp.int32))
    selected = strict | (tie & (rank <= (k - c_strict)))
    x_new = jnp.where(selected[:, None], x * fitness[:, None], 0.0)
    emask_new = emask & selected[src] & selected[dst]
    return x_new, selected, emask_new


def readout_full(x, sel, batch, nb):
    w = sel.astype(x.dtype)
    cnt = jax.ops.segment_sum(w, batch, num_segments=nb, indices_are_sorted=True)
    mean = jax.ops.segment_sum(x * w[:, None], batch, num_segments=nb,
                               indices_are_sorted=True)
    mean = mean / jnp.maximum(cnt, 1.0)[:, None]
    xm = jnp.where(sel[:, None], x, NEG)
    mx = jax.ops.segment_max(xm, batch, num_segments=nb, indices_are_sorted=True)
    mx = jnp.where(mx <= NEG / 2, 0.0, mx)
    return jnp.concatenate([mean, mx], axis=-1)


def kernel(atom_idx, charge_idx, degree_idx, edge_index, batch, sub_fp, fingerprint, emb_atom, emb_charge, emb_degree, proj_W1, proj_b1, proj_W2, proj_b2, fp_W, fp_b, cat_W0, cat_as0, cat_ad0, cat_b0, cat_p0, cat_pb0, cat_W1, cat_as1, cat_ad1, cat_b1, cat_p1, cat_pb1, cat_W2, cat_as2, cat_ad2, cat_b2, cat_p2, cat_pb2, fpc_W0, fpc_as0, fpc_ad0, fpc_b0, fpc_p0, fpc_pb0, fpc_W1, fpc_as1, fpc_ad1, fpc_b1, fpc_p1, fpc_pb1, fpc_W2, fpc_as2, fpc_ad2, fpc_b2, fpc_p2, fpc_pb2, lin1_W, lin1_b, lin2_W, lin2_b):
    src0 = edge_index[0].astype(jnp.int32)
    dst0 = edge_index[1].astype(jnp.int32)
    batch = batch.astype(jnp.int32)
    N = batch.shape[0]

    order = jnp.argsort(dst0)
    src = src0[order]
    dst = dst0[order]

    x_cat0 = embed_proj(atom_idx, charge_idx, degree_idx, emb_atom, emb_charge,
                        emb_degree, proj_W1, proj_b1, proj_W2, proj_b2)

    params = {
        'cat': [(cat_W0, cat_as0, cat_ad0, cat_b0, cat_p0, cat_pb0),
                (cat_W1, cat_as1, cat_ad1, cat_b1, cat_p1, cat_pb1),
                (cat_W2, cat_as2, cat_ad2, cat_b2, cat_p2, cat_pb2)],
        'fpc': [(fpc_W0, fpc_as0, fpc_ad0, fpc_b0, fpc_p0, fpc_pb0),
                (fpc_W1, fpc_as1, fpc_ad1, fpc_b1, fpc_p1, fpc_pb1),
                (fpc_W2, fpc_as2, fpc_ad2, fpc_b2, fpc_p2, fpc_pb2)],
    }

    def run_path(x, prefix):
        alive = jnp.ones((N,), bool)
        emask = jnp.ones(src.shape, bool)
        xs = None
        k = N
        for i in range(3):
            W, a_s, a_d, b, pw, pb = params[prefix][i]
            x = gat_full(x, src, dst, emask, W, a_s, a_d, b)
            x = jnp.where(alive[:, None], x, 0.0)
            k = max(1, int(k * 0.5))
            x, alive, emask = pool_full(x, alive, src, dst, emask, pw, pb, k)
            r = readout_full(x, alive, batch, B_GR)
            xs = r if xs is None else xs + r
        return xs

    xs_cat = run_path(x_cat0, 'cat')
    xs_fp = run_path(sub_fp, 'fpc')
    fp_emb = mm(fingerprint, fp_W, fp_b, bm=64)
    x = jnp.concatenate([fp_emb, xs_cat, xs_fp], axis=-1)
    x = mm(x, lin1_W, lin1_b, act='relu', bm=64)
    lw = jnp.pad(lin2_W, ((0, 0), (0, 127)))
    lb = jnp.pad(lin2_b, (0, 127))
    return mm(x, lw, lb, bm=64)[:, :1]


# speedup vs baseline: 2.8469x; 2.2610x over previous
import functools
import jax
import jax.numpy as jnp
from jax import lax
from jax.experimental import pallas as pl
from jax.experimental.pallas import tpu as pltpu
from jax.experimental.pallas import tpu_sc as plsc

NEG = -1e9
B_GR = 64
H = 512

NC, NS = 2, 16
NW = NC * NS
SUBR = 128
NSUB = 3 * NW
NP = SUBR * NSUB
IB = 512
CH = 32


def _lrelu(x):
    return jnp.where(x >= 0, x, 0.2 * x)


def _mm_body(x_ref, w_ref, b_ref, o_ref, *, act):
    acc = jnp.dot(x_ref[...], w_ref[...], preferred_element_type=jnp.float32)
    acc = acc + b_ref[...]
    if act == 'relu':
        acc = jnp.maximum(acc, 0.0)
    elif act == 'sigmoid':
        acc = jax.nn.sigmoid(acc)
    o_ref[...] = acc


def mm(x, w, b=None, act=None, bm=512):
    M, K = x.shape
    N = w.shape[1]
    if b is None:
        b = jnp.zeros((N,), jnp.float32)
    Mp = ((M + bm - 1) // bm) * bm
    if Mp != M:
        x = jnp.pad(x, ((0, Mp - M), (0, 0)))
    out = pl.pallas_call(
        functools.partial(_mm_body, act=act),
        grid=(Mp // bm,),
        in_specs=[
            pl.BlockSpec((bm, K), lambda i: (i, 0)),
            pl.BlockSpec((K, N), lambda i: (0, 0)),
            pl.BlockSpec((1, N), lambda i: (0, 0)),
        ],
        out_specs=pl.BlockSpec((bm, N), lambda i: (i, 0)),
        out_shape=jax.ShapeDtypeStruct((Mp, N), jnp.float32),
    )(x, w, b.reshape(1, N))
    return out[:M] if Mp != M else out


def _emb_body(ai_ref, ci_ref, di_ref, tab_ref, w1_ref, b1_ref, w2_ref, b2_ref,
              o_ref):
    bm = ai_ref.shape[2]
    ai = ai_ref[0, 0]
    ci = ci_ref[0, 0]
    di = di_ref[0, 0]
    col = jax.lax.broadcasted_iota(jnp.int32, (bm, 384), 1)
    oh = ((col == ai[:, None]) | (col == ci[:, None] + 128)
          | (col == di[:, None] + 256)).astype(jnp.float32)
    embs = jnp.dot(oh, tab_ref[...], preferred_element_type=jnp.float32)
    h = jnp.maximum(jnp.dot(embs, w1_ref[...],
                            preferred_element_type=jnp.float32) + b1_ref[...], 0.0)
    o_ref[...] = jnp.dot(h, w2_ref[...],
                         preferred_element_type=jnp.float32) + b2_ref[...]


def embed_proj(atom_idx, charge_idx, degree_idx, emb_atom, emb_charge,
               emb_degree, W1, b1, W2, b2, bm=512):
    N = atom_idx.shape[0]
    Np = ((N + bm - 1) // bm) * bm
    ai = jnp.pad(atom_idx.astype(jnp.int32), (0, Np - N)).reshape(Np // bm, 1, bm)
    ci = jnp.pad(charge_idx.astype(jnp.int32), (0, Np - N)).reshape(Np // bm, 1, bm)
    di = jnp.pad(degree_idx.astype(jnp.int32), (0, Np - N)).reshape(Np // bm, 1, bm)
    tab = jnp.zeros((384, 384), jnp.float32)
    tab = tab.at[0:100, 0:128].set(emb_atom)
    tab = tab.at[128:144, 128:256].set(emb_charge)
    tab = tab.at[256:264, 256:384].set(emb_degree)
    D = W2.shape[1]
    out = pl.pallas_call(
        _emb_body,
        grid=(Np // bm,),
        in_specs=[
            pl.BlockSpec((1, 1, bm), lambda i: (i, 0, 0)),
            pl.BlockSpec((1, 1, bm), lambda i: (i, 0, 0)),
            pl.BlockSpec((1, 1, bm), lambda i: (i, 0, 0)),
            pl.BlockSpec((384, 384), lambda i: (0, 0)),
            pl.BlockSpec((384, 128), lambda i: (0, 0)),
            pl.BlockSpec((1, 128), lambda i: (0, 0)),
            pl.BlockSpec((128, D), lambda i: (0, 0)),
            pl.BlockSpec((1, D), lambda i: (0, 0)),
        ],
        out_specs=pl.BlockSpec((bm, D), lambda i: (i, 0)),
        out_shape=jax.ShapeDtypeStruct((Np, D), jnp.float32),
    )(ai, ci, di, tab, W1, b1.reshape(1, -1), W2, b2.reshape(1, -1))
    return out[:N]


def _gat_edge_body(h_hbm, ssrc_hbm, sdst_hbm, c_hbm, src_hbm, dst_hbm, off_hbm,
                   out_hbm, den_hbm,
                   ssrc_v, sdst_v, off_v, c_v, src_v, dst_v, rows_v,
                   acc, dacc, sem):
    wid = lax.axis_index("s") * NC + lax.axis_index("c")
    pltpu.sync_copy(ssrc_hbm, ssrc_v)
    pltpu.sync_copy(sdst_hbm, sdst_v)
    pltpu.sync_copy(off_hbm, off_v)
    pltpu.sync_copy(c_hbm, c_v)
    cvec = c_v[...]

    offrow = off_v[wid, pl.ds(0, 16)]
    lo0, lo1, lo2, lo3 = offrow[0], offrow[1], offrow[2], offrow[3]

    def sub_body(t, _):
        s_idx = 3 * wid + t
        base = s_idx * SUBR
        lo = jnp.where(t == 0, lo0, jnp.where(t == 1, lo1, lo2))
        hi = jnp.where(t == 0, lo1, jnp.where(t == 1, lo2, lo3))

        def zero_body(r, _):
            for u in range(H // 16):
                acc[r, pl.ds(16 * u, 16)] = jnp.zeros((16,), jnp.float32)
            for u in range(8):
                dacc[r, pl.ds(16 * u, 16)] = jnp.zeros((16,), jnp.float32)
            return 0

        lax.fori_loop(0, SUBR, zero_body, 0)

        def block_body(bk, _):
            pltpu.sync_copy(src_hbm.at[pl.ds(bk * IB, IB)], src_v)
            pltpu.sync_copy(dst_hbm.at[pl.ds(bk * IB, IB)], dst_v)

            def chunk_body(q, _):
                off = q * CH
                pltpu.async_copy(h_hbm.at[src_v.at[pl.ds(off, CH)]],
                                 rows_v, sem).wait()
                for g in range(CH // 16):
                    eo = off + g * 16
                    s16 = src_v[pl.ds(eo, 16)]
                    d16 = dst_v[pl.ds(eo, 16)]
                    a = (plsc.load_gather(ssrc_v, [s16])
                         + plsc.load_gather(sdst_v, [d16]))
                    alpha = jnp.where(a >= 0, a, 0.2 * a)
                    e = jnp.exp(alpha - cvec)
                    valid = (d16 >= base) & (d16 < base + SUBR)
                    e = jnp.where(valid, e, 0.0)
                    for j in range(16):
                        ej = e[j]

                        @pl.when(ej > 0.0)
                        def _():
                            dl = d16[j] - base
                            for u in range(H // 16):
                                sl = pl.ds(16 * u, 16)
                                acc[dl, sl] = (acc[dl, sl]
                                               + ej * rows_v[g * 16 + j, sl])
                            d0 = pl.ds(0, 16)
                            dacc[dl, d0] = dacc[dl, d0] + ej
                return 0

            lax.fori_loop(0, IB // CH, chunk_body, 0)
            return 0

        lax.fori_loop(lo // IB, (hi + IB - 1) // IB, block_body, 0)
        pltpu.sync_copy(acc, out_hbm.at[pl.ds(base, SUBR)])
        pltpu.sync_copy(dacc, den_hbm.at[pl.ds(base, SUBR)])
        return 0

    lax.fori_loop(0, 3, sub_body, 0)


_gat_edge = pl.kernel(
    _gat_edge_body,
    out_type=[jax.ShapeDtypeStruct((NP, H), jnp.float32),
              jax.ShapeDtypeStruct((NP, 128), jnp.float32)],
    mesh=plsc.VectorSubcoreMesh(core_axis_name="c", subcore_axis_name="s"),
    compiler_params=pltpu.CompilerParams(needs_layout_passes=False),
    scratch_types=[
        pltpu.VMEM((NP,), jnp.float32),
        pltpu.VMEM((NP,), jnp.float32),
        pltpu.VMEM((NW, 16), jnp.int32),
        pltpu.VMEM((16,), jnp.float32),
        pltpu.VMEM((IB,), jnp.int32),
        pltpu.VMEM((IB,), jnp.int32),
        pltpu.VMEM((CH, H), jnp.float32),
        pltpu.VMEM((SUBR, H), jnp.float32),
        pltpu.VMEM((SUBR, 128), jnp.float32),
        pltpu.SemaphoreType.DMA,
    ],
)


def _epi_body(m_ref, d_ref, b_ref, o_ref):
    den = d_ref[:, 0:1] + 1e-16
    o_ref[...] = jnp.maximum(m_ref[...] / den + b_ref[...], 0.0)


def gat_epilogue(msum, den, b, bm=512):
    M = msum.shape[0]
    return pl.pallas_call(
        _epi_body,
        grid=(M // bm,),
        in_specs=[
            pl.BlockSpec((bm, H), lambda i: (i, 0)),
            pl.BlockSpec((bm, 128), lambda i: (i, 0)),
            pl.BlockSpec((1, H), lambda i: (0, 0)),
        ],
        out_specs=pl.BlockSpec((bm, H), lambda i: (i, 0)),
        out_shape=jax.ShapeDtypeStruct((M, H), jnp.float32),
    )(msum, den, b.reshape(1, H))


def _readout_body(g0_ref, g1_ref, bt_ref, x_ref, sel_ref, sum_ref, cnt_ref,
                  mx_ref):
    i = pl.program_id(0)

    @pl.when(i == 0)
    def _():
        sum_ref[...] = jnp.zeros_like(sum_ref)
        cnt_ref[...] = jnp.zeros_like(cnt_ref)
        mx_ref[...] = jnp.full_like(mx_ref, NEG)

    bt = bt_ref[0, 0]
    selv = sel_ref[0, 0]
    bm = bt.shape[0]
    x = x_ref[...] * selv[:, None]
    oh = ((bt[:, None] == jax.lax.broadcasted_iota(jnp.int32, (bm, B_GR), 1))
          .astype(jnp.float32)) * selv[:, None]
    sum_ref[...] += jax.lax.dot_general(oh, x, (((0,), (0,)), ((), ())),
                                        preferred_element_type=jnp.float32)
    cnt_ref[...] += jax.lax.dot_general(
        oh, jnp.ones((bm, 128), jnp.float32), (((0,), (0,)), ((), ())),
        preferred_element_type=jnp.float32)
    xm = jnp.where(selv[:, None] > 0, x_ref[...], NEG)
    g0 = g0_ref[i]
    g1 = g1_ref[i]

    def gbody(g, _):
        m = jnp.max(jnp.where(bt[:, None] == g, xm, NEG), axis=0, keepdims=True)
        mx_ref[pl.ds(g, 1), :] = jnp.maximum(mx_ref[pl.ds(g, 1), :], m)
        return 0

    lax.fori_loop(g0, g1 + 1, gbody, 0)


def readout(x, sel, batch, bm=512):
    M = x.shape[0]
    nblk = M // bm
    btb = batch.reshape(nblk, 1, bm)
    selb = sel.astype(jnp.float32).reshape(nblk, 1, bm)
    g0 = btb[:, 0, 0].astype(jnp.int32)
    g1 = btb[:, 0, -1].astype(jnp.int32)
    sums, cnt, mx = pl.pallas_call(
        _readout_body,
        grid_spec=pltpu.PrefetchScalarGridSpec(
            num_scalar_prefetch=2,
            grid=(nblk,),
            in_specs=[
                pl.BlockSpec((1, 1, bm), lambda i, g0, g1: (i, 0, 0)),
                pl.BlockSpec((bm, H), lambda i, g0, g1: (i, 0)),
                pl.BlockSpec((1, 1, bm), lambda i, g0, g1: (i, 0, 0)),
            ],
            out_specs=[
                pl.BlockSpec((B_GR, H), lambda i, g0, g1: (0, 0)),
                pl.BlockSpec((B_GR, 128), lambda i, g0, g1: (0, 0)),
                pl.BlockSpec((B_GR, H), lambda i, g0, g1: (0, 0)),
            ],
        ),
        out_shape=[jax.ShapeDtypeStruct((B_GR, H), jnp.float32),
                   jax.ShapeDtypeStruct((B_GR, 128), jnp.float32),
                   jax.ShapeDtypeStruct((B_GR, H), jnp.float32)],
    )(g0, g1, btb, x, selb)
    cnt = cnt[:, 0]
    mean = sums / jnp.maximum(cnt, 1.0)[:, None]
    mx = jnp.where(mx <= NEG / 2, 0.0, mx)
    return jnp.concatenate([mean, mx], axis=-1)


def pool_select(x, fitness, alive, k):
    fb = jax.lax.bitcast_convert_type(fitness, jnp.int32)
    key = jnp.where(alive, fb, jnp.int32(-1))
    lo = jnp.int32(-2)
    hi = jnp.int32(0x40000000)

    def body(i, lohi):
        lo, hi = lohi
        mid = lo + (hi - lo) // 2
        cnt = jnp.sum((key > mid).astype(jnp.int32))
        return jnp.where(cnt >= k, mid + 1, lo), jnp.where(cnt >= k, hi, mid)

    lo, hi = jax.lax.fori_loop(0, 32, body, (lo, hi))
    T = lo
    strict = key > T
    c_strict = jnp.sum(strict.astype(jnp.int32))
    tie = key == T
    rank = jnp.cumsum(tie.astype(jnp.int32))
    selected = strict | (tie & (rank <= (k - c_strict)))
    x_new = jnp.where(selected[:, None], x * fitness[:, None], 0.0)
    return x_new, selected


def kernel(atom_idx, charge_idx, degree_idx, edge_index, batch, sub_fp, fingerprint, emb_atom, emb_charge, emb_degree, proj_W1, proj_b1, proj_W2, proj_b2, fp_W, fp_b, cat_W0, cat_as0, cat_ad0, cat_b0, cat_p0, cat_pb0, cat_W1, cat_as1, cat_ad1, cat_b1, cat_p1, cat_pb1, cat_W2, cat_as2, cat_ad2, cat_b2, cat_p2, cat_pb2, fpc_W0, fpc_as0, fpc_ad0, fpc_b0, fpc_p0, fpc_pb0, fpc_W1, fpc_as1, fpc_ad1, fpc_b1, fpc_p1, fpc_pb1, fpc_W2, fpc_as2, fpc_ad2, fpc_b2, fpc_p2, fpc_pb2, lin1_W, lin1_b, lin2_W, lin2_b):
    src0 = edge_index[0].astype(jnp.int32)
    dst0 = edge_index[1].astype(jnp.int32)
    N = batch.shape[0]
    E = src0.shape[0]

    order = jnp.argsort(dst0)
    Ep = ((E + IB - 1) // IB) * IB
    srcs = jnp.pad(src0[order], (0, Ep - E))
    dsts = jnp.pad(dst0[order], (0, Ep - E), constant_values=NP)
    bounds = jnp.arange(NSUB + 1, dtype=jnp.int32) * SUBR
    offs = jnp.searchsorted(dsts[:E], bounds).astype(jnp.int32)
    widx = jnp.arange(NW)[:, None] * 3 + jnp.arange(4)[None, :]
    offs = jnp.pad(offs[widx], ((0, 0), (0, 12)))
    batchp = jnp.pad(batch.astype(jnp.int32), (0, NP - N), constant_values=B_GR - 1)

    x_cat0 = embed_proj(atom_idx, charge_idx, degree_idx, emb_atom, emb_charge,
                        emb_degree, proj_W1, proj_b1, proj_W2, proj_b2)

    params = {
        'cat': [(cat_W0, cat_as0, cat_ad0, cat_b0, cat_p0, cat_pb0),
                (cat_W1, cat_as1, cat_ad1, cat_b1, cat_p1, cat_pb1),
                (cat_W2, cat_as2, cat_ad2, cat_b2, cat_p2, cat_pb2)],
        'fpc': [(fpc_W0, fpc_as0, fpc_ad0, fpc_b0, fpc_p0, fpc_pb0),
                (fpc_W1, fpc_as1, fpc_ad1, fpc_b1, fpc_p1, fpc_pb1),
                (fpc_W2, fpc_as2, fpc_ad2, fpc_b2, fpc_p2, fpc_pb2)],
    }

    def run_path(x, prefix):
        alive = jnp.arange(NP) < N
        xs = None
        k = N
        for i in range(3):
            W, a_s, a_d, b, pw, pb = params[prefix][i]
            h = mm(x, W)
            aw = jnp.stack([a_s, a_d], axis=1)
            s = mm(h, jnp.pad(aw, ((0, 0), (0, 126))))
            s_src = jnp.where(alive, s[:, 0], NEG)
            s_dst = jnp.where(alive, s[:, 1], NEG)
            c = _lrelu(jnp.max(s_src) + jnp.max(s_dst))
            c16 = jnp.broadcast_to(c, (16,))
            msum, den = _gat_edge(h, s_src, s_dst, c16, srcs, dsts, offs)
            x = gat_epilogue(msum, den, b)
            fit = mm(x, jnp.pad(pw[:, None], ((0, 0), (0, 127))),
                     jnp.full((128,), pb), act='sigmoid')[:, 0]
            k = max(1, int(k * 0.5))
            x, alive = pool_select(x, fit, alive, k)
            r = readout(x, alive, batchp)
            xs = r if xs is None else xs + r
        return xs

    xs_cat = run_path(jnp.pad(x_cat0, ((0, NP - N), (0, 0))), 'cat')
    xs_fp = run_path(jnp.pad(sub_fp, ((0, NP - N), (0, 0))), 'fpc')
    fp_emb = mm(fingerprint, fp_W, fp_b, bm=64)
    x = jnp.concatenate([fp_emb, xs_cat, xs_fp], axis=-1)
    x = mm(x, lin1_W, lin1_b, act='relu', bm=64)
    lw = jnp.pad(lin2_W, ((0, 0), (0, 127)))
    lb = jnp.pad(lin2_b, (0, 127))
    return mm(x, lw, lb, bm=64)[:, :1]
